# baseline (speedup 1.0000x reference)
import jax
import jax.numpy as jnp
from jax import lax
from jax.experimental import pallas as pl
from jax.experimental.pallas import tpu as pltpu
from jax.experimental.pallas import tpu_sc as plsc

B = 16384
CTX = 50
D = 64
HID = 128

NC = 2
NS = 16
NW = NC * NS

SAMPLES_PER_W = B // NW
NBUF = 8


TBLOCK = 1024


def _repack_body(xt_ref, out_ref):
    x = xt_ref[...]
    xx = jnp.concatenate([x, x], axis=0)
    out_ref[...] = xx.T


def _repack_table(table_t):
    v = table_t.shape[1]
    return pl.pallas_call(
        _repack_body,
        grid=(pl.cdiv(v, TBLOCK),),
        in_specs=[pl.BlockSpec((D, TBLOCK), lambda i: (0, i))],
        out_specs=pl.BlockSpec((TBLOCK, 2 * D), lambda i: (i, 0)),
        out_shape=jax.ShapeDtypeStruct((v, 2 * D), jnp.float32),
    )(table_t)


def _pool_body(idx_hbm, table_hbm, out_hbm, idx_v, acc_v, *rest):
    rows_bufs, sems = rest[:NBUF], rest[NBUF:]
    wid = lax.axis_index("s") * NC + lax.axis_index("c")
    base = wid * SAMPLES_PER_W

    pltpu.sync_copy(idx_hbm.at[pl.ds(base, SAMPLES_PER_W)], idx_v)

    for b in range(NBUF):
        pltpu.async_copy(table_hbm.at[idx_v.at[b]], rows_bufs[b], sems[b])

    def group(i, carry):
        for b in range(NBUF):
            g = NBUF * i + b
            rows_v, sem = rows_bufs[b], sems[b]
            pltpu.make_async_copy(
                table_hbm.at[idx_v.at[g]], rows_v, sem).wait()
            for seg in range(D // 16):
                v = rows_v[0, pl.ds(16 * seg, 16)]
                for r in range(1, CTX):
                    v = v + rows_v[r, pl.ds(16 * seg, 16)]
                acc_v[g, pl.ds(16 * seg, 16)] = v

            @pl.when(g + NBUF < SAMPLES_PER_W)
            def _():
                pltpu.async_copy(
                    table_hbm.at[idx_v.at[g + NBUF]], rows_v, sem)
        return carry

    lax.fori_loop(0, SAMPLES_PER_W // NBUF, group, 0)
    pltpu.sync_copy(acc_v, out_hbm.at[pl.ds(base, SAMPLES_PER_W)])


def _pooled_embeddings(idx, table):
    kern = pl.kernel(
        _pool_body,
        out_type=jax.ShapeDtypeStruct((B, D), jnp.float32),
        mesh=plsc.VectorSubcoreMesh(
            core_axis_name="c", subcore_axis_name="s",
            num_cores=NC, num_subcores=NS,
        ),
        scratch_types=(
            [
                pltpu.VMEM((SAMPLES_PER_W, CTX), jnp.int32),
                pltpu.VMEM((SAMPLES_PER_W, D), jnp.float32),
            ]
            + [pltpu.VMEM((CTX, 2 * D), jnp.float32)] * NBUF
            + [pltpu.SemaphoreType.DMA] * NBUF
        ),
        compiler_params=pltpu.CompilerParams(use_tc_tiling_on_sc=False),
    )
    return kern(idx, table)


def _mlp_body(x_ref, w1_ref, b1_ref, w2_ref, b2_ref, o_ref):
    h = jnp.dot(x_ref[...], w1_ref[...], preferred_element_type=jnp.float32)
    h = jnp.maximum(h + b1_ref[...], 0.0)
    logits = jnp.dot(h, w2_ref[...], preferred_element_type=jnp.float32)
    logits = logits + b2_ref[...]
    m = jnp.max(logits, axis=1, keepdims=True)
    lse = jnp.log(jnp.sum(jnp.exp(logits - m), axis=1, keepdims=True)) + m
    o_ref[...] = logits - lse


def _mlp(embeds, W1, b1, W2, b2):
    bs = 2048
    return pl.pallas_call(
        _mlp_body,
        grid=(B // bs,),
        in_specs=[
            pl.BlockSpec((bs, D), lambda i: (i, 0)),
            pl.BlockSpec((D, HID), lambda i: (0, 0)),
            pl.BlockSpec((1, HID), lambda i: (0, 0)),
            pl.BlockSpec((HID, 2), lambda i: (0, 0)),
            pl.BlockSpec((1, 2), lambda i: (0, 0)),
        ],
        out_specs=pl.BlockSpec((bs, 2), lambda i: (i, 0)),
        out_shape=jax.ShapeDtypeStruct((B, 2), jnp.float32),
    )(embeds, W1, b1.reshape(1, HID), W2, b2.reshape(1, 2))


@jax.jit
def kernel(inputs, table, W1, b1, W2, b2):
    table128 = _repack_table(table.T)
    embeds = _pooled_embeddings(inputs.astype(jnp.int32), table128)
    return _mlp(embeds, W1, b1, W2, b2)


# speedup vs baseline: 1.6866x; 1.6866x over previous
import jax
import jax.numpy as jnp
from jax import lax
from jax.experimental import pallas as pl
from jax.experimental.pallas import tpu as pltpu
from jax.experimental.pallas import tpu_sc as plsc

B = 16384
CTX = 50
D = 64
HID = 128

NC = 2
NS = 16
NW = NC * NS

SAMPLES_PER_W = B // NW
NBUF = 8


TBLOCK = 8192


def _repack_body(xt_ref, out_ref):
    x = xt_ref[...]
    xx = jnp.concatenate([x, x], axis=0)
    out_ref[...] = xx.T


def _repack_table(table_t):
    v = table_t.shape[1]
    return pl.pallas_call(
        _repack_body,
        grid=(pl.cdiv(v, TBLOCK),),
        in_specs=[pl.BlockSpec((D, TBLOCK), lambda i: (0, i))],
        out_specs=pl.BlockSpec((TBLOCK, 2 * D), lambda i: (i, 0)),
        out_shape=jax.ShapeDtypeStruct((v, 2 * D), jnp.float32),
    )(table_t)


def _pool_body(idx_hbm, table_hbm, out_hbm, idx_v, acc_v, *rest):
    rows_bufs, sems = rest[:NBUF], rest[NBUF:]
    wid = lax.axis_index("s") * NC + lax.axis_index("c")
    base = wid * SAMPLES_PER_W

    pltpu.sync_copy(idx_hbm.at[pl.ds(base, SAMPLES_PER_W)], idx_v)

    for b in range(NBUF):
        pltpu.async_copy(table_hbm.at[idx_v.at[b]], rows_bufs[b], sems[b])

    def group(i, carry):
        for b in range(NBUF):
            g = NBUF * i + b
            rows_v, sem = rows_bufs[b], sems[b]
            pltpu.make_async_copy(
                table_hbm.at[idx_v.at[g]], rows_v, sem).wait()
            for seg in range(D // 16):
                v = rows_v[0, pl.ds(16 * seg, 16)]
                for r in range(1, CTX):
                    v = v + rows_v[r, pl.ds(16 * seg, 16)]
                acc_v[g, pl.ds(16 * seg, 16)] = v

            @pl.when(g + NBUF < SAMPLES_PER_W)
            def _():
                pltpu.async_copy(
                    table_hbm.at[idx_v.at[g + NBUF]], rows_v, sem)
        return carry

    lax.fori_loop(0, SAMPLES_PER_W // NBUF, group, 0)
    pltpu.sync_copy(acc_v, out_hbm.at[pl.ds(base, SAMPLES_PER_W)])


def _pooled_embeddings(idx, table):
    kern = pl.kernel(
        _pool_body,
        out_type=jax.ShapeDtypeStruct((B, D), jnp.float32),
        mesh=plsc.VectorSubcoreMesh(
            core_axis_name="c", subcore_axis_name="s",
            num_cores=NC, num_subcores=NS,
        ),
        scratch_types=(
            [
                pltpu.VMEM((SAMPLES_PER_W, CTX), jnp.int32),
                pltpu.VMEM((SAMPLES_PER_W, D), jnp.float32),
            ]
            + [pltpu.VMEM((CTX, 2 * D), jnp.float32)] * NBUF
            + [pltpu.SemaphoreType.DMA] * NBUF
        ),
        compiler_params=pltpu.CompilerParams(use_tc_tiling_on_sc=False),
    )
    return kern(idx, table)


def _mlp_body(x_ref, w1_ref, b1_ref, w2_ref, b2_ref, o_ref):
    h = jnp.dot(x_ref[...], w1_ref[...], preferred_element_type=jnp.float32)
    h = jnp.maximum(h + b1_ref[...], 0.0)
    logits = jnp.dot(h, w2_ref[...], preferred_element_type=jnp.float32)
    logits = logits + b2_ref[...]
    m = jnp.max(logits, axis=1, keepdims=True)
    lse = jnp.log(jnp.sum(jnp.exp(logits - m), axis=1, keepdims=True)) + m
    o_ref[...] = logits - lse


def _mlp(embeds, W1, b1, W2, b2):
    bs = 2048
    return pl.pallas_call(
        _mlp_body,
        grid=(B // bs,),
        in_specs=[
            pl.BlockSpec((bs, D), lambda i: (i, 0)),
            pl.BlockSpec((D, HID), lambda i: (0, 0)),
            pl.BlockSpec((1, HID), lambda i: (0, 0)),
            pl.BlockSpec((HID, 2), lambda i: (0, 0)),
            pl.BlockSpec((1, 2), lambda i: (0, 0)),
        ],
        out_specs=pl.BlockSpec((bs, 2), lambda i: (i, 0)),
        out_shape=jax.ShapeDtypeStruct((B, 2), jnp.float32),
    )(embeds, W1, b1.reshape(1, HID), W2, b2.reshape(1, 2))


@jax.jit
def kernel(inputs, table, W1, b1, W2, b2):
    table128 = _repack_table(table.T)
    embeds = _pooled_embeddings(inputs.astype(jnp.int32), table128)
    return _mlp(embeds, W1, b1, W2, b2)


# speedup vs baseline: 1.7423x; 1.0330x over previous
import jax
import jax.numpy as jnp
from jax import lax
from jax.experimental import pallas as pl
from jax.experimental.pallas import tpu as pltpu
from jax.experimental.pallas import tpu_sc as plsc

B = 16384
CTX = 50
D = 64
HID = 128

NC = 2
NS = 16
NW = NC * NS

SAMPLES_PER_W = B // NW
NBUF = 8


TBLOCK = 16384


def _repack_body(xt_ref, out_ref):
    x = xt_ref[...]
    xx = jnp.concatenate([x, x], axis=0)
    out_ref[...] = xx.T


def _repack_table(table_t):
    v = table_t.shape[1]
    return pl.pallas_call(
        _repack_body,
        grid=(pl.cdiv(v, TBLOCK),),
        in_specs=[pl.BlockSpec((D, TBLOCK), lambda i: (0, i))],
        out_specs=pl.BlockSpec((TBLOCK, 2 * D), lambda i: (i, 0)),
        out_shape=jax.ShapeDtypeStruct((v, 2 * D), jnp.float32),
    )(table_t)


def _pool_body(idx_hbm, table_hbm, out_hbm, idx_v, acc_v, *rest):
    rows_bufs, sems = rest[:NBUF], rest[NBUF:]
    wid = lax.axis_index("s") * NC + lax.axis_index("c")
    base = wid * SAMPLES_PER_W

    pltpu.sync_copy(idx_hbm.at[pl.ds(base, SAMPLES_PER_W)], idx_v)

    for b in range(NBUF):
        pltpu.async_copy(table_hbm.at[idx_v.at[b]], rows_bufs[b], sems[b])

    def group(i, carry):
        for b in range(NBUF):
            g = NBUF * i + b
            rows_v, sem = rows_bufs[b], sems[b]
            pltpu.make_async_copy(
                table_hbm.at[idx_v.at[g]], rows_v, sem).wait()
            for seg in range(D // 16):
                v = rows_v[0, pl.ds(16 * seg, 16)]
                for r in range(1, CTX):
                    v = v + rows_v[r, pl.ds(16 * seg, 16)]
                acc_v[g, pl.ds(16 * seg, 16)] = v

            @pl.when(g + NBUF < SAMPLES_PER_W)
            def _():
                pltpu.async_copy(
                    table_hbm.at[idx_v.at[g + NBUF]], rows_v, sem)
        return carry

    lax.fori_loop(0, SAMPLES_PER_W // NBUF, group, 0)
    pltpu.sync_copy(acc_v, out_hbm.at[pl.ds(base, SAMPLES_PER_W)])


def _pooled_embeddings(idx, table):
    kern = pl.kernel(
        _pool_body,
        out_type=jax.ShapeDtypeStruct((B, D), jnp.float32),
        mesh=plsc.VectorSubcoreMesh(
            core_axis_name="c", subcore_axis_name="s",
            num_cores=NC, num_subcores=NS,
        ),
        scratch_types=(
            [
                pltpu.VMEM((SAMPLES_PER_W, CTX), jnp.int32),
                pltpu.VMEM((SAMPLES_PER_W, D), jnp.float32),
            ]
            + [pltpu.VMEM((CTX, 2 * D), jnp.float32)] * NBUF
            + [pltpu.SemaphoreType.DMA] * NBUF
        ),
        compiler_params=pltpu.CompilerParams(use_tc_tiling_on_sc=False),
    )
    return kern(idx, table)


def _mlp_body(x_ref, w1_ref, b1_ref, w2_ref, b2_ref, o_ref):
    h = jnp.dot(x_ref[...], w1_ref[...], preferred_element_type=jnp.float32)
    h = jnp.maximum(h + b1_ref[...], 0.0)
    logits = jnp.dot(h, w2_ref[...], preferred_element_type=jnp.float32)
    logits = logits + b2_ref[...]
    m = jnp.max(logits, axis=1, keepdims=True)
    lse = jnp.log(jnp.sum(jnp.exp(logits - m), axis=1, keepdims=True)) + m
    o_ref[...] = logits - lse


def _mlp(embeds, W1, b1, W2, b2):
    bs = 2048
    return pl.pallas_call(
        _mlp_body,
        grid=(B // bs,),
        in_specs=[
            pl.BlockSpec((bs, D), lambda i: (i, 0)),
            pl.BlockSpec((D, HID), lambda i: (0, 0)),
            pl.BlockSpec((1, HID), lambda i: (0, 0)),
            pl.BlockSpec((HID, 2), lambda i: (0, 0)),
            pl.BlockSpec((1, 2), lambda i: (0, 0)),
        ],
        out_specs=pl.BlockSpec((bs, 2), lambda i: (i, 0)),
        out_shape=jax.ShapeDtypeStruct((B, 2), jnp.float32),
    )(embeds, W1, b1.reshape(1, HID), W2, b2.reshape(1, 2))


@jax.jit
def kernel(inputs, table, W1, b1, W2, b2):
    table128 = _repack_table(table.T)
    embeds = _pooled_embeddings(inputs.astype(jnp.int32), table128)
    return _mlp(embeds, W1, b1, W2, b2)


# speedup vs baseline: 1.7549x; 1.0073x over previous
import jax
import jax.numpy as jnp
from jax import lax
from jax.experimental import pallas as pl
from jax.experimental.pallas import tpu as pltpu
from jax.experimental.pallas import tpu_sc as plsc

B = 16384
CTX = 50
D = 64
HID = 128

NC = 2
NS = 16
NW = NC * NS

SAMPLES_PER_W = B // NW
NBUF = 8


TBLOCK = 32768


def _repack_body(xt_ref, out_ref):
    x = xt_ref[...]
    xx = jnp.concatenate([x, x], axis=0)
    out_ref[...] = xx.T


def _repack_table(table_t):
    v = table_t.shape[1]
    return pl.pallas_call(
        _repack_body,
        grid=(pl.cdiv(v, TBLOCK),),
        in_specs=[pl.BlockSpec((D, TBLOCK), lambda i: (0, i))],
        out_specs=pl.BlockSpec((TBLOCK, 2 * D), lambda i: (i, 0)),
        out_shape=jax.ShapeDtypeStruct((v, 2 * D), jnp.float32),
    )(table_t)


def _pool_body(idx_hbm, table_hbm, out_hbm, idx_v, acc_v, *rest):
    rows_bufs, sems = rest[:NBUF], rest[NBUF:]
    wid = lax.axis_index("s") * NC + lax.axis_index("c")
    base = wid * SAMPLES_PER_W

    pltpu.sync_copy(idx_hbm.at[pl.ds(base, SAMPLES_PER_W)], idx_v)

    for b in range(NBUF):
        pltpu.async_copy(table_hbm.at[idx_v.at[b]], rows_bufs[b], sems[b])

    def group(i, carry):
        for b in range(NBUF):
            g = NBUF * i + b
            rows_v, sem = rows_bufs[b], sems[b]
            pltpu.make_async_copy(
                table_hbm.at[idx_v.at[g]], rows_v, sem).wait()
            for seg in range(D // 16):
                v = rows_v[0, pl.ds(16 * seg, 16)]
                for r in range(1, CTX):
                    v = v + rows_v[r, pl.ds(16 * seg, 16)]
                acc_v[g, pl.ds(16 * seg, 16)] = v

            @pl.when(g + NBUF < SAMPLES_PER_W)
            def _():
                pltpu.async_copy(
                    table_hbm.at[idx_v.at[g + NBUF]], rows_v, sem)
        return carry

    lax.fori_loop(0, SAMPLES_PER_W // NBUF, group, 0)
    pltpu.sync_copy(acc_v, out_hbm.at[pl.ds(base, SAMPLES_PER_W)])


def _pooled_embeddings(idx, table):
    kern = pl.kernel(
        _pool_body,
        out_type=jax.ShapeDtypeStruct((B, D), jnp.float32),
        mesh=plsc.VectorSubcoreMesh(
            core_axis_name="c", subcore_axis_name="s",
            num_cores=NC, num_subcores=NS,
        ),
        scratch_types=(
            [
                pltpu.VMEM((SAMPLES_PER_W, CTX), jnp.int32),
                pltpu.VMEM((SAMPLES_PER_W, D), jnp.float32),
            ]
            + [pltpu.VMEM((CTX, 2 * D), jnp.float32)] * NBUF
            + [pltpu.SemaphoreType.DMA] * NBUF
        ),
        compiler_params=pltpu.CompilerParams(use_tc_tiling_on_sc=False),
    )
    return kern(idx, table)


def _mlp_body(x_ref, w1_ref, b1_ref, w2_ref, b2_ref, o_ref):
    h = jnp.dot(x_ref[...], w1_ref[...], preferred_element_type=jnp.float32)
    h = jnp.maximum(h + b1_ref[...], 0.0)
    logits = jnp.dot(h, w2_ref[...], preferred_element_type=jnp.float32)
    logits = logits + b2_ref[...]
    m = jnp.max(logits, axis=1, keepdims=True)
    lse = jnp.log(jnp.sum(jnp.exp(logits - m), axis=1, keepdims=True)) + m
    o_ref[...] = logits - lse


def _mlp(embeds, W1, b1, W2, b2):
    bs = 2048
    return pl.pallas_call(
        _mlp_body,
        grid=(B // bs,),
        in_specs=[
            pl.BlockSpec((bs, D), lambda i: (i, 0)),
            pl.BlockSpec((D, HID), lambda i: (0, 0)),
            pl.BlockSpec((1, HID), lambda i: (0, 0)),
            pl.BlockSpec((HID, 2), lambda i: (0, 0)),
            pl.BlockSpec((1, 2), lambda i: (0, 0)),
        ],
        out_specs=pl.BlockSpec((bs, 2), lambda i: (i, 0)),
        out_shape=jax.ShapeDtypeStruct((B, 2), jnp.float32),
    )(embeds, W1, b1.reshape(1, HID), W2, b2.reshape(1, 2))


@jax.jit
def kernel(inputs, table, W1, b1, W2, b2):
    table128 = _repack_table(table.T)
    embeds = _pooled_embeddings(inputs.astype(jnp.int32), table128)
    return _mlp(embeds, W1, b1, W2, b2)
